# speedup vs baseline: 2.4122x; 2.4122x over previous
import jax
import jax.numpy as jnp
from jax.experimental import pallas as pl
from jax.experimental.pallas import tpu as pltpu

_NS = 128
_NV = 32
_G = 256
_B = 2000


def _body(feats_ref, batch_ref, W1_ref, b1_ref, W2_ref, b2_ref, Wf3_ref,
          forces_ref, acc_ref):
    i = pl.program_id(0)
    feats = feats_ref[...]
    scal = feats[:, :_NS]
    h = scal @ W1_ref[...] + b1_ref[...]
    h = h * jax.nn.sigmoid(h)
    e = h @ W2_ref[...] + b2_ref[...]
    vecs = feats[:, _NS:]
    forces_ref[...] = vecs @ Wf3_ref[...]

    b = batch_ref[0, 0, :]
    oh = (jax.lax.broadcasted_iota(jnp.int32, (_G, _B), 0)
          == b[None, :]).astype(jnp.float32)
    stacked = jnp.concatenate([e, jnp.ones_like(e)], axis=1)
    partial = oh @ stacked

    @pl.when(i == 0)
    def _():
        acc_ref[...] = jnp.zeros_like(acc_ref)
    acc_ref[...] += partial


def kernel(node_feats, batch, W1, b1, W2, b2, Wf):
    n, feat_dim = node_feats.shape
    nsteps = n // _B
    batch32 = batch.astype(jnp.int32).reshape(nsteps, 1, _B)
    wf3 = (Wf[:, None, None] * jnp.eye(3, dtype=Wf.dtype)).reshape(3 * _NV, 3)

    forces, acc = pl.pallas_call(
        _body,
        grid=(nsteps,),
        in_specs=[
            pl.BlockSpec((_B, feat_dim), lambda i: (i, 0)),
            pl.BlockSpec((1, 1, _B), lambda i: (i, 0, 0)),
            pl.BlockSpec((_NS, 64), lambda i: (0, 0)),
            pl.BlockSpec((1, 64), lambda i: (0, 0)),
            pl.BlockSpec((64, 1), lambda i: (0, 0)),
            pl.BlockSpec((1, 1), lambda i: (0, 0)),
            pl.BlockSpec((3 * _NV, 3), lambda i: (0, 0)),
        ],
        out_specs=[
            pl.BlockSpec((_B, 3), lambda i: (i, 0)),
            pl.BlockSpec((_G, 2), lambda i: (0, 0)),
        ],
        out_shape=[
            jax.ShapeDtypeStruct((n, 3), jnp.float32),
            jax.ShapeDtypeStruct((_G, 2), jnp.float32),
        ],
    )(node_feats, batch32, W1, b1.reshape(1, 64), W2, b2.reshape(1, 1), wf3)

    return acc[:, 0], forces, acc[:, 1]
